# baseline (speedup 1.0000x reference)
import jax
import jax.numpy as jnp
from jax import lax
from jax.experimental import pallas as pl
from jax.experimental.pallas import tpu as pltpu
from jax.experimental.pallas import tpu_sc as plsc

_G = 128
_K = 2
_W = _G * _K

_TR = 2048

_N_SC = 409600


def _sc_part(idx, table, n_sc, dim):
    mesh = plsc.VectorSubcoreMesh(core_axis_name="core",
                                  subcore_axis_name="subcore")

    @pl.kernel(out_type=jax.ShapeDtypeStruct((n_sc, dim), table.dtype),
               mesh=mesh,
               scratch_types=[pltpu.VMEM_SHARED((128, 128), jnp.float32)])
    def gather_kernel(table_hbm, i_hbm, o_hbm, table_sh):
        sid = lax.axis_index("subcore")

        @pl.when(sid == 0)
        def _():
            pltpu.sync_copy(table_hbm, table_sh)

        plsc.subcore_barrier()

        def body(i_vmem, o_vmem):
            for j in range(_K):
                pltpu.sync_copy(table_sh.at[i_vmem.at[j]],
                                o_vmem.at[pl.ds(j * _G, _G)])

        pltpu.emit_pipeline(
            body,
            grid=(n_sc // _W,),
            in_specs=[pl.BlockSpec((_K, _G), index_map=lambda i: (i, 0))],
            out_specs=[pl.BlockSpec((_W, dim), index_map=lambda i: (i, 0))],
            core_axis_name=("core", "subcore"),
            dimension_semantics=(pltpu.PARALLEL,),
        )(i_hbm, o_hbm)

    return gather_kernel(table, idx.reshape(n_sc // _G, _G))


def _tc_body(x_ref, table_ref, o_ref):
    row = lax.broadcasted_iota(jnp.int32, (128, 256), 0)
    for j in range(_TR // 256):
        xj = x_ref[0, j, :].reshape(1, 256)
        onehot_t = (row == xj).astype(jnp.float32)
        o_ref[pl.ds(j * 256, 256), :] = lax.dot_general(
            onehot_t, table_ref[...], (((0,), (0,)), ((), ())),
            preferred_element_type=jnp.float32)


def _tc_part(idx, table, n_tc, dim):
    x3 = idx.reshape(n_tc // _TR, _TR // 256, 256)
    return pl.pallas_call(
        _tc_body,
        grid=(n_tc // _TR,),
        in_specs=[
            pl.BlockSpec((1, _TR // 256, 256), lambda i: (i, 0, 0)),
            pl.BlockSpec((128, 128), lambda i: (0, 0)),
        ],
        out_specs=pl.BlockSpec((_TR, dim), lambda i: (i, 0)),
        out_shape=jax.ShapeDtypeStruct((n_tc, dim), table.dtype),
    )(x3, table)


def kernel(x, I, pe):
    batch, hist = x.shape
    dim = I.shape[1]
    n = batch * hist
    flat = x.reshape(n)
    sc_out = _sc_part(flat[:_N_SC], I, _N_SC, dim)
    tc_out = _tc_part(flat[_N_SC:], I, n - _N_SC, dim)
    return jnp.concatenate([sc_out, tc_out], axis=0).reshape(batch, hist, dim)


# speedup vs baseline: 2.5039x; 2.5039x over previous
import jax
import jax.numpy as jnp
from jax.experimental import pallas as pl
from jax.experimental.pallas import tpu as pltpu
from jax.experimental.pallas import tpu_sc as plsc

_G = 128
_K = 2
_W = _G * _K


def kernel(x, I, pe):
    batch, hist = x.shape
    dim = I.shape[1]
    n = batch * hist
    idx = x.reshape(n // _G, _G)

    mesh = plsc.VectorSubcoreMesh(core_axis_name="core",
                                  subcore_axis_name="subcore")

    @pl.kernel(out_type=jax.ShapeDtypeStruct((n, dim), I.dtype), mesh=mesh,
               scratch_types=[pltpu.VMEM_SHARED((128, 128), jnp.float32)])
    def gather_kernel(table_hbm, i_hbm, o_hbm, table_sh):
        sid = jax.lax.axis_index("subcore")

        @pl.when(sid == 0)
        def _():
            pltpu.sync_copy(table_hbm, table_sh)

        plsc.subcore_barrier()

        def body(i_vmem, o_vmem):
            def go(sem):
                copies = [
                    pltpu.async_copy(table_sh.at[i_vmem.at[j]],
                                     o_vmem.at[pl.ds(j * _G, _G)], sem)
                    for j in range(_K)
                ]
                for c in copies:
                    c.wait()

            pl.run_scoped(go, pltpu.SemaphoreType.DMA)

        pltpu.emit_pipeline(
            body,
            grid=(n // _W,),
            in_specs=[pl.BlockSpec((_K, _G), index_map=lambda i: (i, 0))],
            out_specs=[pl.BlockSpec((_W, dim), index_map=lambda i: (i, 0))],
            core_axis_name=("core", "subcore"),
            dimension_semantics=(pltpu.PARALLEL,),
        )(i_hbm, o_hbm)

    out = gather_kernel(I, idx)
    return out.reshape(batch, hist, dim)
